# baseline (speedup 1.0000x reference)
import jax
import jax.numpy as jnp
from jax.experimental import pallas as pl
from jax.experimental.pallas import tpu as pltpu

_N_NODES = 100000
_N_GRAPHS = 256
_CHUNK = 5000
_N_CHUNKS = _N_NODES // _CHUNK


def _fused_body(x_ref, ids_ref, W1_ref, b1_ref, W2_ref, b2_ref, out_ref):
    W1 = W1_ref[...].astype(jnp.bfloat16)
    b1v = b1_ref[...].reshape(1, -1)
    W2 = W2_ref[...].astype(jnp.bfloat16)
    ids_all = ids_ref[...]
    giota = jax.lax.broadcasted_iota(jnp.int32, (_N_GRAPHS, _CHUNK), 0)

    acc = jnp.zeros((_N_GRAPHS, 2), jnp.float32)
    for j in range(_N_CHUNKS):
        x = x_ref[pl.ds(j * _CHUNK, _CHUNK), :].astype(jnp.bfloat16)
        h = jnp.dot(x, W1, preferred_element_type=jnp.float32)
        h = jnp.maximum(h + b1v, 0.0).astype(jnp.bfloat16)
        y = jnp.dot(h, W2, preferred_element_type=jnp.float32)

        ids = jax.lax.slice(ids_all, (j * _CHUNK,),
                            ((j + 1) * _CHUNK,)).reshape(1, _CHUNK)
        onehot = (giota == ids).astype(jnp.bfloat16)
        yo = jnp.concatenate([y, jnp.ones_like(y)],
                             axis=1).astype(jnp.bfloat16)
        acc = acc + jnp.dot(onehot, yo,
                            preferred_element_type=jnp.float32)

    s = acc[:, 0:1]
    c = acc[:, 1:2]
    out_ref[...] = (s + c * b2_ref[...].reshape(1, 1)) / jnp.maximum(c, 1.0)


def kernel(x, W1, b1, W2, b2, batch):
    out = pl.pallas_call(
        _fused_body,
        out_shape=jax.ShapeDtypeStruct((_N_GRAPHS, 1), jnp.float32),
    )(x, batch.astype(jnp.int32), W1, b1, W2, b2)
    return out


# speedup vs baseline: 1.0965x; 1.0965x over previous
import jax
import jax.numpy as jnp
from jax.experimental import pallas as pl
from jax.experimental.pallas import tpu as pltpu

_N_NODES = 100000
_N_GRAPHS = 256
_BLK = 12800
_GRID = 8
_HALF = _BLK // 2


def _fused_body(x_ref, ids_ref, W1_ref, b1_ref, W2_ref, b2_ref, out_ref,
                acc_ref):
    i = pl.program_id(0)

    @pl.when(i == 0)
    def _init():
        acc_ref[...] = jnp.zeros_like(acc_ref)

    W1 = W1_ref[...].astype(jnp.bfloat16)
    W2 = W2_ref[...].astype(jnp.bfloat16)
    b1v = b1_ref[...]
    deltas = []
    for half in range(2):
        base = half * _HALF
        row = (jax.lax.broadcasted_iota(jnp.int32, (_HALF, 1), 0)
               + (i * _BLK + base))
        x = jnp.where(row < _N_NODES, x_ref[pl.ds(base, _HALF), :],
                      0.0).astype(jnp.bfloat16)
        h = jnp.dot(x, W1, preferred_element_type=jnp.float32)
        h = jnp.maximum(h + b1v, 0.0).astype(jnp.bfloat16)
        y = jnp.dot(h, W2, preferred_element_type=jnp.float32)

        ids = ids_ref[0, :, pl.ds(base, _HALF)]
        onehot = (jax.lax.broadcasted_iota(jnp.int32, (_N_GRAPHS, _HALF), 0)
                  == ids).astype(jnp.bfloat16)
        yo = jnp.concatenate([y, jnp.ones_like(y)],
                             axis=1).astype(jnp.bfloat16)
        deltas.append(jnp.dot(onehot, yo,
                              preferred_element_type=jnp.float32))
    acc_ref[...] += deltas[0] + deltas[1]

    @pl.when(i == _GRID - 1)
    def _finish():
        s = acc_ref[:, 0:1]
        c = acc_ref[:, 1:2]
        out_ref[...] = (s + c * b2_ref[...].reshape(1, 1)) / jnp.maximum(c, 1.0)


def kernel(x, W1, b1, W2, b2, batch):
    ids = jnp.pad(batch.astype(jnp.int32), (0, _GRID * _BLK - _N_NODES),
                  constant_values=_N_GRAPHS).reshape(_GRID, 1, _BLK)
    out = pl.pallas_call(
        _fused_body,
        grid=(_GRID,),
        in_specs=[
            pl.BlockSpec((_BLK, x.shape[1]), lambda i: (i, 0)),
            pl.BlockSpec((1, 1, _BLK), lambda i: (i, 0, 0)),
            pl.BlockSpec(W1.shape, lambda i: (0, 0)),
            pl.BlockSpec(b1.shape, lambda i: (0,)),
            pl.BlockSpec(W2.shape, lambda i: (0, 0)),
            pl.BlockSpec(b2.shape, lambda i: (0,)),
        ],
        out_specs=pl.BlockSpec((_N_GRAPHS, 1), lambda i: (0, 0)),
        out_shape=jax.ShapeDtypeStruct((_N_GRAPHS, 1), jnp.float32),
        scratch_shapes=[pltpu.VMEM((_N_GRAPHS, 2), jnp.float32)],
        compiler_params=pltpu.CompilerParams(
            dimension_semantics=("arbitrary",)),
    )(x, ids, W1, b1, W2, b2)
    return out


# speedup vs baseline: 1.2288x; 1.1207x over previous
import jax
import jax.numpy as jnp
from jax.experimental import pallas as pl
from jax.experimental.pallas import tpu as pltpu

_N_NODES = 100000
_N_GRAPHS = 256
_BLK = 12800
_GRID = 8
_HALF = _BLK // 2


def _fused_body(x_ref, ids_ref, W1_ref, b1_ref, W2_ref, b2_ref, out_ref,
                acc_ref):
    i = pl.program_id(0)

    @pl.when(i == 0)
    def _init():
        acc_ref[...] = jnp.zeros_like(acc_ref)

    row = jax.lax.broadcasted_iota(jnp.int32, (_BLK, 1), 0) + i * _BLK
    x = jnp.where(row < _N_NODES, x_ref[...], 0.0).astype(jnp.bfloat16)
    h = jnp.dot(x, W1_ref[...].astype(jnp.bfloat16),
                preferred_element_type=jnp.float32)
    h = jnp.maximum(h + b1_ref[...], 0.0).astype(jnp.bfloat16)
    y = jnp.dot(h, W2_ref[...].astype(jnp.bfloat16),
                preferred_element_type=jnp.float32)

    ids = ids_ref[0]
    onehot = (jax.lax.broadcasted_iota(jnp.int32, (_N_GRAPHS, _BLK), 0)
              == ids).astype(jnp.bfloat16)
    yo = jnp.concatenate([y, jnp.ones_like(y)],
                         axis=1).astype(jnp.bfloat16)
    acc_ref[...] += jnp.dot(onehot, yo,
                            preferred_element_type=jnp.float32)

    @pl.when(i == _GRID - 1)
    def _finish():
        s = acc_ref[:, 0:1]
        c = acc_ref[:, 1:2]
        out_ref[...] = (s + c * b2_ref[...].reshape(1, 1)) / jnp.maximum(c, 1.0)


def kernel(x, W1, b1, W2, b2, batch):
    ids = jnp.pad(batch.astype(jnp.int32), (0, _GRID * _BLK - _N_NODES),
                  constant_values=_N_GRAPHS).reshape(_GRID, 1, _BLK)
    out = pl.pallas_call(
        _fused_body,
        grid=(_GRID,),
        in_specs=[
            pl.BlockSpec((_BLK, x.shape[1]), lambda i: (i, 0)),
            pl.BlockSpec((1, 1, _BLK), lambda i: (i, 0, 0)),
            pl.BlockSpec(W1.shape, lambda i: (0, 0)),
            pl.BlockSpec(b1.shape, lambda i: (0,)),
            pl.BlockSpec(W2.shape, lambda i: (0, 0)),
            pl.BlockSpec(b2.shape, lambda i: (0,)),
        ],
        out_specs=pl.BlockSpec((_N_GRAPHS, 1), lambda i: (0, 0)),
        out_shape=jax.ShapeDtypeStruct((_N_GRAPHS, 1), jnp.float32),
        scratch_shapes=[pltpu.VMEM((_N_GRAPHS, 2), jnp.float32)],
        compiler_params=pltpu.CompilerParams(
            dimension_semantics=("arbitrary",)),
    )(x, ids, W1, b1, W2, b2)
    return out
